# Initial kernel scaffold; baseline (speedup 1.0000x reference)
import jax
import jax.numpy as jnp
from jax.experimental import pallas as pl


def kernel(h_t, Wg, W1, b1, W2, b2):
    raise NotImplementedError("write your pallas kernel here")


# speedup vs baseline: 1.1230x; 1.1230x over previous
import jax
import jax.numpy as jnp
from jax.experimental import pallas as pl
from jax.experimental.pallas import tpu as pltpu

_N = 4096
_D = 1024
_FF = 4096
_E = 8
_FC = 1024
_TC = 1024
_RB = 1024


def _router_kernel(x_ref, wg_ref, w8_ref):
    logits = jnp.dot(x_ref[...], wg_ref[...], preferred_element_type=jnp.float32)
    iota = jax.lax.broadcasted_iota(jnp.int32, logits.shape, 1)
    m1 = jnp.max(logits, axis=1, keepdims=True)
    i1 = jnp.argmax(logits, axis=1)
    l2 = jnp.where(iota == i1[:, None], jnp.float32(-1e30), logits)
    m2 = jnp.max(l2, axis=1, keepdims=True)
    i2 = jnp.argmax(l2, axis=1)
    r = jnp.exp(m2 - m1)
    wa = 1.0 / (1.0 + r)
    wb = 1.0 - wa
    w8 = jnp.where(iota == i1[:, None], wa,
                   jnp.where(iota == i2[:, None], wb, 0.0))
    w8_ref[...] = w8


def _moe_kernel(w_ref, xb_ref, w1_ref, b1_ref, w2_ref, b2_ref, out_ref):
    e = pl.program_id(0)
    f = pl.program_id(1)

    @pl.when(jnp.logical_and(e == 0, f == 0))
    def _init():
        out_ref[...] = jnp.zeros_like(out_ref)

    w1 = w1_ref[0].astype(jnp.bfloat16)
    w2 = w2_ref[0].astype(jnp.bfloat16)
    b1 = b1_ref[0]
    b2 = jnp.where(f == 0, b2_ref[0], jnp.zeros_like(b2_ref[0]))
    for i in range(_N // _TC):
        rows = pl.ds(i * _TC, _TC)
        acc = jnp.dot(xb_ref[rows, :], w1, preferred_element_type=jnp.float32)
        a = jax.nn.gelu(acc + b1)
        y = jnp.dot(a.astype(jnp.bfloat16), w2, preferred_element_type=jnp.float32)
        we = w_ref[0, rows, :]
        out_ref[rows, :] += we * (y + b2)


def _router(x, wg):
    return pl.pallas_call(
        _router_kernel,
        grid=(_N // _RB,),
        in_specs=[
            pl.BlockSpec((_RB, _D), lambda i: (i, 0)),
            pl.BlockSpec((_D, _E), lambda i: (0, 0)),
        ],
        out_specs=pl.BlockSpec((_RB, _E), lambda i: (i, 0)),
        out_shape=jax.ShapeDtypeStruct((_N, _E), jnp.float32),
    )(x, wg)


def _moe(w_in, xb, W1, b1, W2, b2):
    grid = (_E, _FF // _FC)
    return pl.pallas_call(
        _moe_kernel,
        grid=grid,
        in_specs=[
            pl.BlockSpec((1, _N, 1), lambda e, f: (e, 0, 0)),
            pl.BlockSpec((_N, _D), lambda e, f: (0, 0)),
            pl.BlockSpec((1, _D, _FC), lambda e, f: (e, 0, f)),
            pl.BlockSpec((1, 1, _FC), lambda e, f: (e, 0, f)),
            pl.BlockSpec((1, _FC, _D), lambda e, f: (e, f, 0)),
            pl.BlockSpec((1, 1, _D), lambda e, f: (e, 0, 0)),
        ],
        out_specs=pl.BlockSpec((_N, _D), lambda e, f: (0, 0)),
        out_shape=jax.ShapeDtypeStruct((_N, _D), jnp.float32),
        compiler_params=pltpu.CompilerParams(
            dimension_semantics=("arbitrary", "arbitrary"),
        ),
    )(w_in, xb, W1, b1.reshape(_E, 1, _FF), W2, b2.reshape(_E, 1, _D))


def kernel(h_t, Wg, W1, b1, W2, b2):
    B, S, D = h_t.shape
    x = h_t.reshape(B * S, D)
    w8 = _router(x, Wg)
    w_in = w8.T.reshape(_E, _N, 1)
    xb = x.astype(jnp.bfloat16)
    out = _moe(w_in, xb, W1, b1, W2, b2)
    return out.reshape(B, S, D)
